# baseline (speedup 1.0000x reference)
import functools

import jax
import jax.numpy as jnp
from jax import lax
from jax.experimental import pallas as pl
from jax.experimental.pallas import tpu as pltpu
from jax.experimental.pallas import tpu_sc as plsc

B = 16384
N = 20
EMB = 32
NC = 2
NS = 16
NW = NC * NS
BPW = B // NW
C = 64
NCH = BPW // C
NG = (C * N) // 128
ROWS_PER_W = (BPW * N) // 128


def _sc_pred_body(users_hbm, bundles_hbm, remb, demb, out_hbm,
                  uidx, bidx, urows, drows, pred, pscr, sem):
    cid = lax.axis_index("c")
    sid = lax.axis_index("s")
    wid = sid * NC + cid
    ubase = pl.multiple_of(wid * BPW, BPW)

    pltpu.sync_copy(users_hbm.at[pl.ds(ubase, BPW)], uidx)
    pltpu.sync_copy(bundles_hbm.at[pl.ds(wid * ROWS_PER_W, ROWS_PER_W)], bidx)

    def chunk_body(ch, carry):
        u_off = pl.multiple_of(ch * C, C)
        cps = [pltpu.async_copy(remb.at[uidx.at[pl.ds(u_off, C)]], urows, sem)]
        for g in range(NG):
            cps.append(pltpu.async_copy(
                demb.at[bidx.at[ch * NG + g]],
                drows.at[pl.ds(g * 128, 128)], sem))
        for cp in cps:
            cp.wait()

        iota16 = lax.iota(jnp.int32, 16) * 16

        def blk_body(b4, carry2):
            for uu in range(4):
                u = b4 * 4 + uu
                u0 = urows[u, pl.ds(0, 16)]
                u1 = urows[u, pl.ds(16, 16)]
                for i in range(N):
                    r = u * N + i
                    d0 = drows[r, pl.ds(0, 16)]
                    d1 = drows[r, pl.ds(16, 16)]
                    pscr[pl.ds((uu * N + i) * 16, 16)] = d0 * u0 + d1 * u1
            for k in range(5):
                acc = plsc.load_gather(pscr, [iota16 + (k * 256)])
                for d in range(1, 16):
                    acc = acc + plsc.load_gather(pscr, [iota16 + (k * 256 + d)])
                pred[pl.ds(b4 * 80 + k * 16, 16)] = acc
            return carry2

        lax.fori_loop(0, C // 4, blk_body, 0)
        pltpu.sync_copy(pred, out_hbm.at[pl.ds((ubase + u_off) * N, C * N)])
        return carry

    lax.fori_loop(0, NCH, chunk_body, 0)


_sc_pred = pl.kernel(
    _sc_pred_body,
    out_type=jax.ShapeDtypeStruct((B * N,), jnp.float32),
    mesh=plsc.VectorSubcoreMesh(core_axis_name="c", subcore_axis_name="s"),
    compiler_params=pltpu.CompilerParams(
        needs_layout_passes=False, use_tc_tiling_on_sc=False),
    scratch_types=[
        pltpu.VMEM((BPW,), jnp.int32),
        pltpu.VMEM((ROWS_PER_W, 128), jnp.int32),
        pltpu.VMEM((C, EMB), jnp.float32),
        pltpu.VMEM((C * N, EMB), jnp.float32),
        pltpu.VMEM((C * N,), jnp.float32),
        pltpu.VMEM((80 * 16,), jnp.float32),
        pltpu.SemaphoreType.DMA,
    ],
)


def _loss_body(pred_ref, w_ref, out_ref):
    pred = pred_ref[...]
    w = w_ref[...]
    pos = pred[:, 0:1]
    negs = pred[:, 1:]
    loss = -jax.nn.log_sigmoid(pos - negs) * w
    out_ref[0, 0] = jnp.sum(loss) / (B * (N - 1))


_tc_loss = pl.pallas_call(
    _loss_body,
    out_shape=jax.ShapeDtypeStruct((1, 1), jnp.float32),
    out_specs=pl.BlockSpec(memory_space=pltpu.SMEM),
)


@jax.jit
def kernel(users, bundles, weights, reviewer_emb, diner_emb):
    users_flat = users.reshape(B).astype(jnp.int32)
    bundles2d = bundles.reshape((B * N) // 128, 128).astype(jnp.int32)
    pred = _sc_pred(users_flat, bundles2d, reviewer_emb, diner_emb)
    loss = _tc_loss(pred.reshape(B, N), weights)
    return loss[0, 0]
